# baseline (speedup 1.0000x reference)
import jax
import jax.numpy as jnp
from jax import lax
from jax.experimental import pallas as pl
from jax.experimental.pallas import tpu as pltpu
from jax.experimental.pallas import tpu_sc as plsc

_DIM = 32
_VOCAB = 1000000
_NPAD = 2048
_B = 16384
_L = 50
_NC = 2
_NS = 16
_NW = _NC * _NS
_BPW = _B // _NW
_WROWS = _VOCAB // _NW
_ZROWS = _NPAD // _NW
_BT = 256

_mesh = plsc.VectorSubcoreMesh(
    core_axis_name="c", subcore_axis_name="s",
    num_cores=_NC, num_subcores=_NS,
)
_sc_params = pltpu.CompilerParams(
    use_tc_tiling_on_sc=False, needs_layout_passes=False)


def _wid():
    return lax.axis_index("s") * _NC + lax.axis_index("c")


def _sel_body(x_ref, m_ref, o_ref):
    xv = x_ref[...]
    mv = m_ref[...][:, :, 0]
    idx = jnp.where(mv, xv, _VOCAB + (xv & (_NPAD - 1)))
    o_ref[...] = idx.T


_CROWS = 625
_NCHUNK = _WROWS // _CROWS


def _pad_body(w_hbm, wz_hbm, buf0, buf1, zv, sr0, sr1):
    wid = _wid()
    base = wid * _WROWS
    bufs = (buf0, buf1)
    sems = (sr0, sr1)

    pltpu.async_copy(w_hbm.at[pl.ds(base, _CROWS)], buf0, sr0)
    for g in range(_NCHUNK):
        pltpu.make_async_copy(
            w_hbm.at[pl.ds(base, _CROWS)], bufs[g % 2], sems[g % 2]).wait()
        if g + 1 < _NCHUNK:
            pltpu.async_copy(
                w_hbm.at[pl.ds(base + (g + 1) * _CROWS, _CROWS)],
                bufs[(g + 1) % 2], sems[(g + 1) % 2])
        pltpu.sync_copy(bufs[g % 2],
                        wz_hbm.at[pl.ds(base + g * _CROWS, _CROWS)])

    for r in range(_ZROWS):
        for h in range(_DIM // 16):
            zv[r, pl.ds(h * 16, 16)] = jnp.zeros((16,), jnp.float32)
    pltpu.sync_copy(zv, wz_hbm.at[pl.ds(_VOCAB + wid * _ZROWS, _ZROWS)])


def _body(idx_hbm, wz_hbm, out_hbm, idx, acc, sem0, sem):
    wid = _wid()
    base = wid * _BPW

    pltpu.sync_copy(idx_hbm.at[:, pl.ds(base, _BPW)], idx)

    pltpu.async_copy(wz_hbm.at[idx.at[0]], acc, sem0).wait()

    for l in range(1, _L):
        pltpu.async_copy(wz_hbm.at[idx.at[l]], acc, sem, add=True)
    for l in range(1, _L):
        pltpu.make_async_copy(wz_hbm.at[idx.at[0]], acc, sem).wait()

    pltpu.sync_copy(acc, out_hbm.at[pl.ds(base, _BPW)])


def kernel(x, mask, W):
    sel = pl.pallas_call(
        _sel_body,
        out_shape=jax.ShapeDtypeStruct((_L, _B), jnp.int32),
        grid=(_B // _BT,),
        in_specs=[
            pl.BlockSpec((_BT, _L), lambda i: (i, 0)),
            pl.BlockSpec((_BT, _L, 1), lambda i: (i, 0, 0)),
        ],
        out_specs=pl.BlockSpec((_L, _BT), lambda i: (0, i)),
    )
    idx_arr = sel(x, mask)

    pad_k = pl.kernel(
        _pad_body,
        out_type=jax.ShapeDtypeStruct((_VOCAB + _NPAD, _DIM), jnp.float32),
        mesh=_mesh,
        compiler_params=_sc_params,
        scratch_types=[
            pltpu.VMEM((_CROWS, _DIM), jnp.float32),
            pltpu.VMEM((_CROWS, _DIM), jnp.float32),
            pltpu.VMEM((_ZROWS, _DIM), jnp.float32),
            pltpu.SemaphoreType.DMA,
            pltpu.SemaphoreType.DMA,
        ],
    )
    wz = pad_k(W)

    k = pl.kernel(
        _body,
        out_type=jax.ShapeDtypeStruct((_B, _DIM), jnp.float32),
        mesh=_mesh,
        compiler_params=_sc_params,
        scratch_types=[
            pltpu.VMEM((_L, _BPW), jnp.int32),
            pltpu.VMEM((_BPW, _DIM), jnp.float32),
            pltpu.SemaphoreType.DMA,
            pltpu.SemaphoreType.DMA,
        ],
    )
    return k(idx_arr, wz)


# speedup vs baseline: 1.7768x; 1.7768x over previous
import jax
import jax.numpy as jnp
from jax import lax
from jax.experimental import pallas as pl
from jax.experimental.pallas import tpu as pltpu
from jax.experimental.pallas import tpu_sc as plsc

_DIM = 32
_VOCAB = 1000000
_NPAD = 2048
_B = 16384
_L = 50
_NC = 2
_NS = 16
_NW = _NC * _NS
_BPW = _B // _NW
_NV = _BPW // 16
_WROWS = _VOCAB // _NW
_ZROWS = _NPAD // _NW

_mesh = plsc.VectorSubcoreMesh(
    core_axis_name="c", subcore_axis_name="s",
    num_cores=_NC, num_subcores=_NS,
)
_sc_params = pltpu.CompilerParams(
    use_tc_tiling_on_sc=False, needs_layout_passes=False)


def _wid():
    return lax.axis_index("s") * _NC + lax.axis_index("c")


_CROWS = 625
_NCHUNK = _WROWS // _CROWS


def _pad_body(w_hbm, wz_hbm, buf0, buf1, zv, sr0, sr1):
    wid = _wid()
    base = wid * _WROWS
    bufs = (buf0, buf1)
    sems = (sr0, sr1)

    pltpu.async_copy(w_hbm.at[pl.ds(base, _CROWS)], buf0, sr0)
    for g in range(_NCHUNK):
        pltpu.make_async_copy(
            w_hbm.at[pl.ds(base, _CROWS)], bufs[g % 2], sems[g % 2]).wait()
        if g + 1 < _NCHUNK:
            pltpu.async_copy(
                w_hbm.at[pl.ds(base + (g + 1) * _CROWS, _CROWS)],
                bufs[(g + 1) % 2], sems[(g + 1) % 2])
        pltpu.sync_copy(bufs[g % 2],
                        wz_hbm.at[pl.ds(base + g * _CROWS, _CROWS)])

    for r in range(_ZROWS):
        for h in range(_DIM // 16):
            zv[r, pl.ds(h * 16, 16)] = jnp.zeros((16,), jnp.float32)
    pltpu.sync_copy(zv, wz_hbm.at[pl.ds(_VOCAB + wid * _ZROWS, _ZROWS)])


def _body(x_hbm, m_hbm, wz_hbm, out_hbm, xb, mb, idx, acc, sem0, sem):
    wid = _wid()
    base = wid * _BPW

    pltpu.sync_copy(x_hbm.at[pl.ds(base, _BPW)], xb)
    pltpu.sync_copy(m_hbm.at[pl.ds(base, _BPW)], mb)

    lane = lax.iota(jnp.int32, 16)

    def build(l, carry):
        col = jnp.full((16,), 0, jnp.int32) + l
        for i in range(_NV):
            row = lane + (i * 16)
            xv = plsc.load_gather(xb, [row, col])
            mv = plsc.load_gather(mb, [row, col])
            idx[l, pl.ds(i * 16, 16)] = jnp.where(
                mv > 0, xv, _VOCAB + (xv & (_NPAD - 1)))
        return carry

    lax.fori_loop(0, _L, build, 0)

    pltpu.async_copy(wz_hbm.at[idx.at[0]], acc, sem0).wait()

    for l in range(1, _L):
        pltpu.async_copy(wz_hbm.at[idx.at[l]], acc, sem, add=True)
    for l in range(1, _L):
        pltpu.make_async_copy(wz_hbm.at[idx.at[0]], acc, sem).wait()

    pltpu.sync_copy(acc, out_hbm.at[pl.ds(base, _BPW)])


def kernel(x, mask, W):
    pad_k = pl.kernel(
        _pad_body,
        out_type=jax.ShapeDtypeStruct((_VOCAB + _NPAD, _DIM), jnp.float32),
        mesh=_mesh,
        compiler_params=_sc_params,
        scratch_types=[
            pltpu.VMEM((_CROWS, _DIM), jnp.float32),
            pltpu.VMEM((_CROWS, _DIM), jnp.float32),
            pltpu.VMEM((_ZROWS, _DIM), jnp.float32),
            pltpu.SemaphoreType.DMA,
            pltpu.SemaphoreType.DMA,
        ],
    )
    wz = pad_k(W)

    m32 = mask[:, :, 0].astype(jnp.int32)

    k = pl.kernel(
        _body,
        out_type=jax.ShapeDtypeStruct((_B, _DIM), jnp.float32),
        mesh=_mesh,
        compiler_params=_sc_params,
        scratch_types=[
            pltpu.VMEM((_BPW, _L), jnp.int32),
            pltpu.VMEM((_BPW, _L), jnp.int32),
            pltpu.VMEM((_L, _BPW), jnp.int32),
            pltpu.VMEM((_BPW, _DIM), jnp.float32),
            pltpu.SemaphoreType.DMA,
            pltpu.SemaphoreType.DMA,
        ],
    )
    return k(x, m32, wz)
